# speedup vs baseline: 1.2701x; 1.2701x over previous
import functools

import jax
import jax.numpy as jnp
from jax import lax
from jax.experimental import pallas as pl
from jax.experimental.pallas import tpu as pltpu
from jax.experimental.pallas import tpu_sc as plsc

D = 128
S = 2048
B = 4
BT = B * S
NC = 2
NS = 16
NW = NC * NS
ROWS_W = BT // NW
CHUNK = 128
NCHUNK = ROWS_W // CHUNK
LANES = 16


def _emb_body(x_hbm, tok_hbm, pos_hbm, out_hbm, idx_v, rows_v, pos_v, sem):
    wid = lax.axis_index("s") * NC + lax.axis_index("c")
    base = wid * ROWS_W
    pbase = lax.rem(base, S)
    pltpu.sync_copy(x_hbm.at[wid], idx_v)
    cps = [
        pltpu.async_copy(
            tok_hbm.at[idx_v.at[k]],
            rows_v.at[pl.ds(k * CHUNK, CHUNK)],
            sem,
        )
        for k in range(NCHUNK)
    ]
    pltpu.sync_copy(pos_hbm.at[pl.ds(pbase, ROWS_W)], pos_v)
    for cp in cps:
        cp.wait()

    def add_row(i, carry):
        for j in range(D // LANES):
            sl = pl.ds(j * LANES, LANES)
            plsc.addupdate(rows_v.at[i, sl], pos_v[i, sl])
        return carry

    lax.fori_loop(0, ROWS_W, add_row, 0)
    pltpu.sync_copy(rows_v, out_hbm.at[pl.ds(base, ROWS_W)])


@jax.jit
def kernel(x, token_table, pos_table):
    xr = x.reshape(NW, NCHUNK, CHUNK).astype(jnp.int32)
    mesh = plsc.VectorSubcoreMesh(core_axis_name="c", subcore_axis_name="s")
    run = functools.partial(
        pl.kernel,
        mesh=mesh,
        out_type=jax.ShapeDtypeStruct((BT, D), jnp.float32),
        scratch_types=[
            pltpu.VMEM((NCHUNK, CHUNK), jnp.int32),
            pltpu.VMEM((ROWS_W, D), jnp.float32),
            pltpu.VMEM((ROWS_W, D), jnp.float32),
            pltpu.SemaphoreType.DMA,
        ],
    )(_emb_body)
    out = run(xr, token_table, pos_table)
    return out.reshape(B, S, D)
